# baseline (speedup 1.0000x reference)
import functools
import math

import jax
import jax.numpy as jnp
from jax import lax
from jax.experimental import pallas as pl
from jax.experimental.pallas import tpu as pltpu
from jax.experimental.pallas import tpu_sc as plsc

_INTERPRET = False

K_TOP = 8
H = 8


def _qk_body(q_ref, kT_ref, wqT_ref, wk_ref, wqiT_ref, bqi_ref, st_ref,
             pos_ref, wkiT_ref, wviT_ref, bki_ref, bvi_ref,
             logits_ref, q8T_ref, posk_ref, posv_ref,
             *, inv_sqrt_d, inv_sqrt_dh):
    q = q_ref[0]
    qh = jnp.dot(q, wqT_ref[...], preferred_element_type=jnp.float32)
    khT = jnp.dot(wk_ref[...], kT_ref[0], preferred_element_type=jnp.float32)
    logits_ref[0] = jnp.dot(qh, khT, preferred_element_type=jnp.float32) * inv_sqrt_d
    q2 = jnp.dot(qh, wqiT_ref[...], preferred_element_type=jnp.float32) + bqi_ref[...]
    T = q2.shape[0]
    q8 = st_ref[...][:, None, :] * (q2 * inv_sqrt_dh)[None, :, :]
    q8T_ref[0] = jnp.transpose(q8.reshape(H * T, q2.shape[1]))
    posk_ref[...] = jnp.dot(pos_ref[...], wkiT_ref[...],
                            preferred_element_type=jnp.float32) + bki_ref[...]
    posv_ref[...] = jnp.dot(pos_ref[...], wviT_ref[...],
                            preferred_element_type=jnp.float32) + bvi_ref[...]


def _fused_body(c_ref, wkiT_ref, wviT_ref, posk_ref, posv_ref,
                q8T_ref, swT_ref, woT_ref, bo_ref,
                out_ref, o_scr, *, mt, t_len, n_mt):
    m = pl.program_id(1)
    CH, C, D = c_ref.shape[1], c_ref.shape[2], c_ref.shape[3]
    T = t_len
    dh = D // H
    x = c_ref[0].reshape(CH * C, D)
    k2 = (jnp.dot(x, wkiT_ref[...], preferred_element_type=jnp.float32)
          .reshape(CH, C, D) + posk_ref[...][None, :, :])
    v2 = (jnp.dot(x, wviT_ref[...], preferred_element_type=jnp.float32)
          .reshape(CH, C, D) + posv_ref[...][None, :, :])
    sc = jnp.dot(k2.reshape(CH * C, D), q8T_ref[0],
                 preferred_element_type=jnp.float32)
    scr = sc.reshape(CH, C, H * T)
    mx = jnp.max(scr, axis=1, keepdims=True)
    e = jnp.exp(scr - mx)
    attn = e / jnp.sum(e, axis=1, keepdims=True)
    os = []
    for h in range(H):
        attn_h = attn[:, :, h * T:(h + 1) * T]
        v2_h = v2[:, :, h * dh:(h + 1) * dh]
        os.append(lax.dot_general(
            attn_h, v2_h, (((1,), (1,)), ((0,), (0,))),
            preferred_element_type=jnp.float32))
    o_scr[pl.ds(m * mt, mt)] = jnp.concatenate(os, axis=2)

    @pl.when(m == n_mt - 1)
    def _():
        sw = swT_ref[0]
        ctx = jnp.sum(o_scr[...] * sw[:, :, None], axis=0)
        out_ref[0] = jnp.dot(ctx, woT_ref[...],
                             preferred_element_type=jnp.float32) + bo_ref[...]


def _lane_shuffle(x, perm):
    return lax.gather(
        x, perm[:, None],
        dimension_numbers=lax.GatherDimensionNumbers(
            offset_dims=(), collapsed_slice_dims=(0,), start_index_map=(0,)),
        slice_sizes=(1,),
        mode=lax.GatherScatterMode.PROMISE_IN_BOUNDS)


def _lane_reduce(x, op, lane):
    for shift in (1, 2, 4, 8):
        perm = jnp.bitwise_and(lane + shift, 15)
        x = op(x, _lane_shuffle(x, perm))
    return x


def _sc_topk(logits2d, n_rows, n_mem, k_top):
    NC, NS = 2, 16
    NW = NC * NS
    rpw = n_rows // NW
    nv = n_mem // 16
    mesh = plsc.VectorSubcoreMesh(core_axis_name="c", subcore_axis_name="s")

    @functools.partial(
        pl.kernel,
        mesh=mesh,
        out_type=jax.ShapeDtypeStruct((n_rows, n_mem), jnp.float32),
        scratch_types=[pltpu.VMEM((n_mem,), jnp.float32),
                       pltpu.VMEM((n_mem,), jnp.float32)],
    )
    def sc_body(logits_hbm, selw_hbm, lrow_v, sel_v):
        wid = lax.axis_index("s") * NC + lax.axis_index("c")
        lane = lax.broadcasted_iota(jnp.int32, (16,), 0)
        zero16 = jnp.zeros((16,), jnp.float32)
        NEG = jnp.float32(-3.0e38)
        for r in range(rpw):
            row = wid * rpw + r
            pltpu.sync_copy(logits_hbm.at[row], lrow_v)
            vals = [lrow_v[pl.ds(j * 16, 16)] for j in range(nv)]
            idxs = [lane + j * 16 for j in range(nv)]
            topv = jnp.full((16,), NEG, jnp.float32)
            topi = jnp.zeros((16,), jnp.int32)
            for kk in range(k_top):
                m = vals[0]
                for j in range(1, nv):
                    m = jnp.maximum(m, vals[j])
                mx = _lane_reduce(m, jnp.maximum, lane)
                cand = jnp.full((16,), jnp.int32(n_mem * 2), jnp.int32)
                for j in range(nv):
                    cand = jnp.minimum(
                        cand, jnp.where(vals[j] == mx, idxs[j],
                                        jnp.int32(n_mem * 2)))
                ami = _lane_reduce(cand, jnp.minimum, lane)
                topv = jnp.where(lane == kk, mx, topv)
                topi = jnp.where(lane == kk, ami, topi)
                for j in range(nv):
                    vals[j] = jnp.where(idxs[j] == ami, NEG, vals[j])
            mall = _lane_reduce(topv, jnp.maximum, lane)
            e = jnp.exp(topv - mall)
            e = jnp.where(lane < k_top, e, jnp.float32(0.0))
            wrow = e / _lane_reduce(e, jnp.add, lane)
            sel = [zero16] * nv
            for kk in range(k_top):
                pk = jnp.full((16,), kk, jnp.int32)
                tb = _lane_shuffle(topi, pk)
                wb = _lane_shuffle(wrow, pk)
                for j in range(nv):
                    sel[j] = jnp.where(idxs[j] == tb, wb, sel[j])
            for j in range(nv):
                sel_v[pl.ds(j * 16, 16)] = sel[j]
            pltpu.sync_copy(sel_v, selw_hbm.at[row])

    return sc_body(logits2d)


def kernel(queries, keys, contents, steps_since_last_write, accumulator,
           Wq, Wk, Wv, in_proj_w, in_proj_b, out_w, out_b):
    B, T, D = queries.shape
    _, M, C, _ = contents.shape
    dh = D // H
    BT = B * T
    f32 = jnp.float32

    Wqi, Wki, Wvi = jnp.split(in_proj_w, 3, axis=0)
    bqi, bki, bvi = jnp.split(in_proj_b, 3)
    WqT, WqiT = Wq.T, Wqi.T
    WkiT, WviT = Wki.T, Wvi.T
    woT = out_w.T
    keysT = keys.transpose(0, 2, 1)

    freqs = jnp.arange(0.0, D, 2.0)
    inv_freq = 10000.0 ** (-freqs / D)
    pos_seq = jnp.arange(C - 1.0, -1.0, -1.0)
    sinusoid = pos_seq[:, None] * inv_freq[None, :]
    pos = jnp.concatenate([jnp.sin(sinusoid), jnp.cos(sinusoid)], axis=-1)

    MT = 8
    n_mt = M // MT

    S = (jnp.arange(D)[:, None] // dh == jnp.arange(H)[None, :]).astype(f32)
    ST = S.T

    logits, q8T, posk, posv = pl.pallas_call(
        functools.partial(_qk_body, inv_sqrt_d=1.0 / math.sqrt(D),
                          inv_sqrt_dh=1.0 / math.sqrt(dh)),
        grid=(B,),
        in_specs=[
            pl.BlockSpec((1, T, D), lambda b: (b, 0, 0)),
            pl.BlockSpec((1, D, M), lambda b: (b, 0, 0)),
            pl.BlockSpec((D, D), lambda b: (0, 0)),
            pl.BlockSpec((D, D), lambda b: (0, 0)),
            pl.BlockSpec((D, D), lambda b: (0, 0)),
            pl.BlockSpec((1, D), lambda b: (0, 0)),
            pl.BlockSpec((H, D), lambda b: (0, 0)),
            pl.BlockSpec((C, D), lambda b: (0, 0)),
            pl.BlockSpec((D, D), lambda b: (0, 0)),
            pl.BlockSpec((D, D), lambda b: (0, 0)),
            pl.BlockSpec((1, D), lambda b: (0, 0)),
            pl.BlockSpec((1, D), lambda b: (0, 0)),
        ],
        out_specs=[
            pl.BlockSpec((1, T, M), lambda b: (b, 0, 0)),
            pl.BlockSpec((1, D, T * H), lambda b: (b, 0, 0)),
            pl.BlockSpec((C, D), lambda b: (0, 0)),
            pl.BlockSpec((C, D), lambda b: (0, 0)),
        ],
        out_shape=[
            jax.ShapeDtypeStruct((B, T, M), f32),
            jax.ShapeDtypeStruct((B, D, T * H), f32),
            jax.ShapeDtypeStruct((C, D), f32),
            jax.ShapeDtypeStruct((C, D), f32),
        ],
        compiler_params=pltpu.CompilerParams(
            dimension_semantics=("arbitrary",)),
        interpret=_INTERPRET,
    )(queries, keysT, WqT, Wk, WqiT, bqi.reshape(1, D), ST, pos,
      WkiT, WviT, bki.reshape(1, D), bvi.reshape(1, D))

    selw = _sc_topk(logits.reshape(BT, M), BT, M, K_TOP)
    swT = selw.reshape(B, T, M).transpose(0, 2, 1)

    out = pl.pallas_call(
        functools.partial(_fused_body, mt=MT, t_len=T, n_mt=n_mt),
        grid=(B, n_mt),
        in_specs=[
            pl.BlockSpec((1, MT, C, D), lambda b, m: (b, m, 0, 0)),
            pl.BlockSpec((D, D), lambda b, m: (0, 0)),
            pl.BlockSpec((D, D), lambda b, m: (0, 0)),
            pl.BlockSpec((C, D), lambda b, m: (0, 0)),
            pl.BlockSpec((C, D), lambda b, m: (0, 0)),
            pl.BlockSpec((1, D, T * H), lambda b, m: (b, 0, 0)),
            pl.BlockSpec((1, M, T), lambda b, m: (b, 0, 0)),
            pl.BlockSpec((D, D), lambda b, m: (0, 0)),
            pl.BlockSpec((1, D), lambda b, m: (0, 0)),
        ],
        out_specs=pl.BlockSpec((1, T, D), lambda b, m: (b, 0, 0)),
        out_shape=jax.ShapeDtypeStruct((B, T, D), f32),
        scratch_shapes=[pltpu.VMEM((M, T, D), f32)],
        compiler_params=pltpu.CompilerParams(
            dimension_semantics=("arbitrary", "arbitrary")),
        interpret=_INTERPRET,
    )(contents.astype(jnp.bfloat16), WkiT.astype(jnp.bfloat16),
      WviT.astype(jnp.bfloat16), posk, posv,
      q8T, swT, woT, out_b.reshape(1, D))

    return out


# speedup vs baseline: 1.2480x; 1.2480x over previous
import functools
import math

import jax
import jax.numpy as jnp
from jax import lax
from jax.experimental import pallas as pl
from jax.experimental.pallas import tpu as pltpu
from jax.experimental.pallas import tpu_sc as plsc

_INTERPRET = False

K_TOP = 8
H = 8


def _qk_body(q_ref, kT_ref, wqT_ref, wk_ref, wqiT_ref, bqi_ref, st_ref,
             pos_ref, wkiT_ref, wviT_ref, bki_ref, bvi_ref,
             logits_ref, q8T_ref, posk_ref, posv_ref,
             *, inv_sqrt_d, inv_sqrt_dh):
    q = q_ref[0]
    qh = jnp.dot(q, wqT_ref[...], preferred_element_type=jnp.float32)
    khT = jnp.dot(wk_ref[...], kT_ref[0], preferred_element_type=jnp.float32)
    logits_ref[0] = jnp.dot(qh, khT, preferred_element_type=jnp.float32) * inv_sqrt_d
    q2 = jnp.dot(qh, wqiT_ref[...], preferred_element_type=jnp.float32) + bqi_ref[...]
    T = q2.shape[0]
    q8 = (q2 * inv_sqrt_dh)[:, None, :] * st_ref[...][None, :, :]
    q8T_ref[0] = jnp.transpose(q8.reshape(T * H, q2.shape[1]))
    posk_ref[...] = jnp.dot(pos_ref[...], wkiT_ref[...],
                            preferred_element_type=jnp.float32) + bki_ref[...]
    posv_ref[...] = jnp.dot(pos_ref[...], wviT_ref[...],
                            preferred_element_type=jnp.float32) + bvi_ref[...]


def _fused_body(c_ref, wkiT_ref, wviT_ref, posk_ref, posv_ref,
                q8T_ref, swT_ref, woT_ref, bo_ref,
                out_ref, ctx_scr, *, mt, t_len, n_mt):
    m = pl.program_id(1)
    CH, C, D = c_ref.shape[1], c_ref.shape[2], c_ref.shape[3]
    T = t_len
    dh = D // H
    x = c_ref[0].reshape(CH * C, D)
    k2 = (jnp.dot(x, wkiT_ref[...], preferred_element_type=jnp.float32)
          .reshape(CH, C, D) + posk_ref[...][None, :, :])
    v2 = (jnp.dot(x, wviT_ref[...], preferred_element_type=jnp.float32)
          .reshape(CH, C, D) + posv_ref[...][None, :, :])
    sc = jnp.dot(k2.reshape(CH * C, D), q8T_ref[0],
                 preferred_element_type=jnp.float32)
    scr = sc.reshape(CH, C, T * H)
    mx = jnp.max(scr, axis=1, keepdims=True)
    e = jnp.exp(scr - mx)
    attn = e / jnp.sum(e, axis=1, keepdims=True)
    allo = lax.dot_general(attn, v2, (((1,), (1,)), ((0,), (0,))),
                           preferred_element_type=jnp.float32)
    alr = allo.reshape(CH, T, H, D)
    o_tile = jnp.concatenate(
        [alr[:, :, h, h * dh:(h + 1) * dh] for h in range(H)], axis=2)
    swm = swT_ref[0]
    contrib = jnp.sum(o_tile * swm[:, :, None], axis=0)

    @pl.when(m == 0)
    def _():
        ctx_scr[...] = jnp.zeros_like(ctx_scr)

    ctx_scr[...] += contrib

    @pl.when(m == n_mt - 1)
    def _():
        out_ref[0] = jnp.dot(ctx_scr[...], woT_ref[...],
                             preferred_element_type=jnp.float32) + bo_ref[...]


def _lane_shuffle(x, perm):
    return lax.gather(
        x, perm[:, None],
        dimension_numbers=lax.GatherDimensionNumbers(
            offset_dims=(), collapsed_slice_dims=(0,), start_index_map=(0,)),
        slice_sizes=(1,),
        mode=lax.GatherScatterMode.PROMISE_IN_BOUNDS)


def _lane_reduce(x, op, lane):
    for shift in (1, 2, 4, 8):
        perm = jnp.bitwise_and(lane + shift, 15)
        x = op(x, _lane_shuffle(x, perm))
    return x


def _sc_topk(logits2d, n_rows, n_mem, k_top):
    NC, NS = 2, 16
    NW = NC * NS
    rpw = n_rows // NW
    nv = n_mem // 16
    mesh = plsc.VectorSubcoreMesh(core_axis_name="c", subcore_axis_name="s")

    @functools.partial(
        pl.kernel,
        mesh=mesh,
        out_type=jax.ShapeDtypeStruct((n_rows, n_mem), jnp.float32),
        scratch_types=[pltpu.VMEM((n_mem,), jnp.float32),
                       pltpu.VMEM((n_mem,), jnp.float32)],
    )
    def sc_body(logits_hbm, selw_hbm, lrow_v, sel_v):
        wid = lax.axis_index("s") * NC + lax.axis_index("c")
        lane = lax.broadcasted_iota(jnp.int32, (16,), 0)
        zero16 = jnp.zeros((16,), jnp.float32)
        NEG = jnp.float32(-3.0e38)
        for r in range(rpw):
            row = wid * rpw + r
            pltpu.sync_copy(logits_hbm.at[row], lrow_v)
            vals = [lrow_v[pl.ds(j * 16, 16)] for j in range(nv)]
            idxs = [lane + j * 16 for j in range(nv)]
            topv = jnp.full((16,), NEG, jnp.float32)
            topi = jnp.zeros((16,), jnp.int32)
            for kk in range(k_top):
                m = vals[0]
                for j in range(1, nv):
                    m = jnp.maximum(m, vals[j])
                mx = _lane_reduce(m, jnp.maximum, lane)
                cand = jnp.full((16,), jnp.int32(n_mem * 2), jnp.int32)
                for j in range(nv):
                    cand = jnp.minimum(
                        cand, jnp.where(vals[j] == mx, idxs[j],
                                        jnp.int32(n_mem * 2)))
                ami = _lane_reduce(cand, jnp.minimum, lane)
                topv = jnp.where(lane == kk, mx, topv)
                topi = jnp.where(lane == kk, ami, topi)
                for j in range(nv):
                    vals[j] = jnp.where(idxs[j] == ami, NEG, vals[j])
            mall = _lane_reduce(topv, jnp.maximum, lane)
            e = jnp.exp(topv - mall)
            e = jnp.where(lane < k_top, e, jnp.float32(0.0))
            wrow = e / _lane_reduce(e, jnp.add, lane)
            sel = [zero16] * nv
            for kk in range(k_top):
                pk = jnp.full((16,), kk, jnp.int32)
                tb = _lane_shuffle(topi, pk)
                wb = _lane_shuffle(wrow, pk)
                for j in range(nv):
                    sel[j] = jnp.where(idxs[j] == tb, wb, sel[j])
            for j in range(nv):
                sel_v[pl.ds(j * 16, 16)] = sel[j]
            pltpu.sync_copy(sel_v, selw_hbm.at[row])

    return sc_body(logits2d)


def kernel(queries, keys, contents, steps_since_last_write, accumulator,
           Wq, Wk, Wv, in_proj_w, in_proj_b, out_w, out_b):
    B, T, D = queries.shape
    _, M, C, _ = contents.shape
    dh = D // H
    BT = B * T
    f32 = jnp.float32

    Wqi, Wki, Wvi = jnp.split(in_proj_w, 3, axis=0)
    bqi, bki, bvi = jnp.split(in_proj_b, 3)
    WqT, WqiT = Wq.T, Wqi.T
    WkiT, WviT = Wki.T, Wvi.T
    woT = out_w.T
    keysT = keys.transpose(0, 2, 1)

    freqs = jnp.arange(0.0, D, 2.0)
    inv_freq = 10000.0 ** (-freqs / D)
    pos_seq = jnp.arange(C - 1.0, -1.0, -1.0)
    sinusoid = pos_seq[:, None] * inv_freq[None, :]
    pos = jnp.concatenate([jnp.sin(sinusoid), jnp.cos(sinusoid)], axis=-1)

    MT = 8
    n_mt = M // MT

    S = (jnp.arange(D)[:, None] // dh == jnp.arange(H)[None, :]).astype(f32)
    ST = S.T

    logits, q8T, posk, posv = pl.pallas_call(
        functools.partial(_qk_body, inv_sqrt_d=1.0 / math.sqrt(D),
                          inv_sqrt_dh=1.0 / math.sqrt(dh)),
        grid=(B,),
        in_specs=[
            pl.BlockSpec((1, T, D), lambda b: (b, 0, 0)),
            pl.BlockSpec((1, D, M), lambda b: (b, 0, 0)),
            pl.BlockSpec((D, D), lambda b: (0, 0)),
            pl.BlockSpec((D, D), lambda b: (0, 0)),
            pl.BlockSpec((D, D), lambda b: (0, 0)),
            pl.BlockSpec((1, D), lambda b: (0, 0)),
            pl.BlockSpec((H, D), lambda b: (0, 0)),
            pl.BlockSpec((C, D), lambda b: (0, 0)),
            pl.BlockSpec((D, D), lambda b: (0, 0)),
            pl.BlockSpec((D, D), lambda b: (0, 0)),
            pl.BlockSpec((1, D), lambda b: (0, 0)),
            pl.BlockSpec((1, D), lambda b: (0, 0)),
        ],
        out_specs=[
            pl.BlockSpec((1, T, M), lambda b: (b, 0, 0)),
            pl.BlockSpec((1, D, T * H), lambda b: (b, 0, 0)),
            pl.BlockSpec((C, D), lambda b: (0, 0)),
            pl.BlockSpec((C, D), lambda b: (0, 0)),
        ],
        out_shape=[
            jax.ShapeDtypeStruct((B, T, M), f32),
            jax.ShapeDtypeStruct((B, D, T * H), f32),
            jax.ShapeDtypeStruct((C, D), f32),
            jax.ShapeDtypeStruct((C, D), f32),
        ],
        compiler_params=pltpu.CompilerParams(
            dimension_semantics=("arbitrary",)),
        interpret=_INTERPRET,
    )(queries, keysT, WqT, Wk, WqiT, bqi.reshape(1, D), ST, pos,
      WkiT, WviT, bki.reshape(1, D), bvi.reshape(1, D))

    selw = _sc_topk(logits.reshape(BT, M), BT, M, K_TOP)
    swT = selw.reshape(B, T, M).transpose(0, 2, 1)

    out = pl.pallas_call(
        functools.partial(_fused_body, mt=MT, t_len=T, n_mt=n_mt),
        grid=(B, n_mt),
        in_specs=[
            pl.BlockSpec((1, MT, C, D), lambda b, m: (b, m, 0, 0)),
            pl.BlockSpec((D, D), lambda b, m: (0, 0)),
            pl.BlockSpec((D, D), lambda b, m: (0, 0)),
            pl.BlockSpec((C, D), lambda b, m: (0, 0)),
            pl.BlockSpec((C, D), lambda b, m: (0, 0)),
            pl.BlockSpec((1, D, T * H), lambda b, m: (b, 0, 0)),
            pl.BlockSpec((1, MT, T), lambda b, m: (b, m, 0)),
            pl.BlockSpec((D, D), lambda b, m: (0, 0)),
            pl.BlockSpec((1, D), lambda b, m: (0, 0)),
        ],
        out_specs=pl.BlockSpec((1, T, D), lambda b, m: (b, 0, 0)),
        out_shape=jax.ShapeDtypeStruct((B, T, D), f32),
        scratch_shapes=[pltpu.VMEM((T, D), f32)],
        compiler_params=pltpu.CompilerParams(
            dimension_semantics=("arbitrary", "arbitrary")),
        interpret=_INTERPRET,
    )(contents, WkiT, WviT, posk, posv,
      q8T, swT, woT, out_b.reshape(1, D))

    return out


# speedup vs baseline: 1.3160x; 1.0545x over previous
import functools
import math

import jax
import jax.numpy as jnp
from jax import lax
from jax.experimental import pallas as pl
from jax.experimental.pallas import tpu as pltpu
from jax.experimental.pallas import tpu_sc as plsc

_INTERPRET = False

K_TOP = 8
H = 8


def _qk_body(q_ref, kT_ref, wqT_ref, wk_ref, wqiT_ref, bqi_ref, st_ref,
             pos_ref, wkiT_ref, wviT_ref, bki_ref, bvi_ref,
             logits_ref, q8T_ref, posk_ref, posv_ref,
             *, inv_sqrt_d, inv_sqrt_dh):
    q = q_ref[0]
    qh = jnp.dot(q, wqT_ref[...], preferred_element_type=jnp.float32)
    khT = jnp.dot(wk_ref[...], kT_ref[0], preferred_element_type=jnp.float32)
    logits_ref[0] = jnp.dot(qh, khT, preferred_element_type=jnp.float32) * inv_sqrt_d
    q2 = jnp.dot(qh, wqiT_ref[...], preferred_element_type=jnp.float32) + bqi_ref[...]
    T = q2.shape[0]
    q8 = (q2 * inv_sqrt_dh)[:, None, :] * st_ref[...][None, :, :]
    q8T_ref[0] = jnp.transpose(q8.reshape(T * H, q2.shape[1]))
    posk_ref[...] = jnp.dot(pos_ref[...], wkiT_ref[...],
                            preferred_element_type=jnp.float32) + bki_ref[...]
    posv_ref[...] = jnp.dot(pos_ref[...], wviT_ref[...],
                            preferred_element_type=jnp.float32) + bvi_ref[...]


def _fused_body(c_ref, wkiT_ref, wviT_ref, posk_ref, posv_ref,
                q8T_ref, swT_ref, woT_ref, bo_ref,
                out_ref, ctx_scr, *, mt, t_len, n_mt):
    m = pl.program_id(1)
    CH, C, D = c_ref.shape[1], c_ref.shape[2], c_ref.shape[3]
    T = t_len
    dh = D // H
    x = c_ref[0].reshape(CH * C, D)
    k2 = (jnp.dot(x, wkiT_ref[...], preferred_element_type=jnp.float32)
          .reshape(CH, C, D) + posk_ref[...][None, :, :])
    v2 = (jnp.dot(x, wviT_ref[...], preferred_element_type=jnp.float32)
          .reshape(CH, C, D) + posv_ref[...][None, :, :])
    sc = jnp.dot(k2.reshape(CH * C, D), q8T_ref[0],
                 preferred_element_type=jnp.float32)
    scr = sc.reshape(CH, C, T * H)
    mx = jnp.max(scr, axis=1, keepdims=True)
    e = jnp.exp(scr - mx)
    attn = e / jnp.sum(e, axis=1, keepdims=True)
    allo = lax.dot_general(attn, v2, (((1,), (1,)), ((0,), (0,))),
                           preferred_element_type=jnp.float32)
    alr = allo.reshape(CH, T, H, D)
    o_tile = jnp.concatenate(
        [alr[:, :, h, h * dh:(h + 1) * dh] for h in range(H)], axis=2)
    swm = swT_ref[0]
    contrib = jnp.sum(o_tile * swm[:, :, None], axis=0)

    @pl.when(m == 0)
    def _():
        ctx_scr[...] = jnp.zeros_like(ctx_scr)

    ctx_scr[...] += contrib

    @pl.when(m == n_mt - 1)
    def _():
        out_ref[0] = jnp.dot(ctx_scr[...], woT_ref[...],
                             preferred_element_type=jnp.float32) + bo_ref[...]


def _lane_shuffle(x, perm):
    return lax.gather(
        x, perm[:, None],
        dimension_numbers=lax.GatherDimensionNumbers(
            offset_dims=(), collapsed_slice_dims=(0,), start_index_map=(0,)),
        slice_sizes=(1,),
        mode=lax.GatherScatterMode.PROMISE_IN_BOUNDS)


def _lane_reduce(x, op, lane):
    for shift in (1, 2, 4, 8):
        perm = jnp.bitwise_and(lane + shift, 15)
        x = op(x, _lane_shuffle(x, perm))
    return x


def _sc_topk(logits2d, n_rows, n_mem, k_top):
    NC, NS = 2, 16
    NW = NC * NS
    rpw = n_rows // NW
    nv = n_mem // 16
    mesh = plsc.VectorSubcoreMesh(core_axis_name="c", subcore_axis_name="s")

    @functools.partial(
        pl.kernel,
        mesh=mesh,
        out_type=jax.ShapeDtypeStruct((n_rows, n_mem), jnp.float32),
        scratch_types=[pltpu.VMEM((n_mem,), jnp.float32),
                       pltpu.VMEM((n_mem,), jnp.float32)],
    )
    def sc_body(logits_hbm, selw_hbm, lrow_v, sel_v):
        wid = lax.axis_index("s") * NC + lax.axis_index("c")
        lane = lax.broadcasted_iota(jnp.int32, (16,), 0)
        zero16 = jnp.zeros((16,), jnp.float32)
        NEG = jnp.float32(-3.0e38)
        for r in range(rpw):
            row = wid * rpw + r
            pltpu.sync_copy(logits_hbm.at[row], lrow_v)
            vals = [lrow_v[pl.ds(j * 16, 16)] for j in range(nv)]
            idxs = [lane + j * 16 for j in range(nv)]
            topv = jnp.full((16,), NEG, jnp.float32)
            topi = jnp.zeros((16,), jnp.int32)
            for kk in range(k_top):
                m = vals[0]
                for j in range(1, nv):
                    m = jnp.maximum(m, vals[j])
                mx = _lane_reduce(m, jnp.maximum, lane)
                cand = jnp.full((16,), jnp.int32(n_mem * 2), jnp.int32)
                for j in range(nv):
                    cand = jnp.minimum(
                        cand, jnp.where(vals[j] == mx, idxs[j],
                                        jnp.int32(n_mem * 2)))
                ami = _lane_reduce(cand, jnp.minimum, lane)
                topv = jnp.where(lane == kk, mx, topv)
                topi = jnp.where(lane == kk, ami, topi)
                for j in range(nv):
                    vals[j] = jnp.where(idxs[j] == ami, NEG, vals[j])
            mall = _lane_reduce(topv, jnp.maximum, lane)
            e = jnp.exp(topv - mall)
            e = jnp.where(lane < k_top, e, jnp.float32(0.0))
            wrow = e / _lane_reduce(e, jnp.add, lane)
            sel = [zero16] * nv
            for kk in range(k_top):
                pk = jnp.full((16,), kk, jnp.int32)
                tb = _lane_shuffle(topi, pk)
                wb = _lane_shuffle(wrow, pk)
                for j in range(nv):
                    sel[j] = jnp.where(idxs[j] == tb, wb, sel[j])
            for j in range(nv):
                sel_v[pl.ds(j * 16, 16)] = sel[j]
            pltpu.sync_copy(sel_v, selw_hbm.at[row])

    return sc_body(logits2d)


def kernel(queries, keys, contents, steps_since_last_write, accumulator,
           Wq, Wk, Wv, in_proj_w, in_proj_b, out_w, out_b):
    B, T, D = queries.shape
    _, M, C, _ = contents.shape
    dh = D // H
    BT = B * T
    f32 = jnp.float32

    Wqi, Wki, Wvi = jnp.split(in_proj_w, 3, axis=0)
    bqi, bki, bvi = jnp.split(in_proj_b, 3)
    WqT, WqiT = Wq.T, Wqi.T
    WkiT, WviT = Wki.T, Wvi.T
    woT = out_w.T
    keysT = keys.transpose(0, 2, 1)

    freqs = jnp.arange(0.0, D, 2.0)
    inv_freq = 10000.0 ** (-freqs / D)
    pos_seq = jnp.arange(C - 1.0, -1.0, -1.0)
    sinusoid = pos_seq[:, None] * inv_freq[None, :]
    pos = jnp.concatenate([jnp.sin(sinusoid), jnp.cos(sinusoid)], axis=-1)

    MT = 16
    n_mt = M // MT

    S = (jnp.arange(D)[:, None] // dh == jnp.arange(H)[None, :]).astype(f32)
    ST = S.T

    logits, q8T, posk, posv = pl.pallas_call(
        functools.partial(_qk_body, inv_sqrt_d=1.0 / math.sqrt(D),
                          inv_sqrt_dh=1.0 / math.sqrt(dh)),
        grid=(B,),
        in_specs=[
            pl.BlockSpec((1, T, D), lambda b: (b, 0, 0)),
            pl.BlockSpec((1, D, M), lambda b: (b, 0, 0)),
            pl.BlockSpec((D, D), lambda b: (0, 0)),
            pl.BlockSpec((D, D), lambda b: (0, 0)),
            pl.BlockSpec((D, D), lambda b: (0, 0)),
            pl.BlockSpec((1, D), lambda b: (0, 0)),
            pl.BlockSpec((H, D), lambda b: (0, 0)),
            pl.BlockSpec((C, D), lambda b: (0, 0)),
            pl.BlockSpec((D, D), lambda b: (0, 0)),
            pl.BlockSpec((D, D), lambda b: (0, 0)),
            pl.BlockSpec((1, D), lambda b: (0, 0)),
            pl.BlockSpec((1, D), lambda b: (0, 0)),
        ],
        out_specs=[
            pl.BlockSpec((1, T, M), lambda b: (b, 0, 0)),
            pl.BlockSpec((1, D, T * H), lambda b: (b, 0, 0)),
            pl.BlockSpec((C, D), lambda b: (0, 0)),
            pl.BlockSpec((C, D), lambda b: (0, 0)),
        ],
        out_shape=[
            jax.ShapeDtypeStruct((B, T, M), f32),
            jax.ShapeDtypeStruct((B, D, T * H), f32),
            jax.ShapeDtypeStruct((C, D), f32),
            jax.ShapeDtypeStruct((C, D), f32),
        ],
        compiler_params=pltpu.CompilerParams(
            dimension_semantics=("arbitrary",)),
        interpret=_INTERPRET,
    )(queries, keysT, WqT, Wk, WqiT, bqi.reshape(1, D), ST, pos,
      WkiT, WviT, bki.reshape(1, D), bvi.reshape(1, D))

    selw = _sc_topk(logits.reshape(BT, M), BT, M, K_TOP)
    swT = selw.reshape(B, T, M).transpose(0, 2, 1)

    out = pl.pallas_call(
        functools.partial(_fused_body, mt=MT, t_len=T, n_mt=n_mt),
        grid=(B, n_mt),
        in_specs=[
            pl.BlockSpec((1, MT, C, D), lambda b, m: (b, m, 0, 0)),
            pl.BlockSpec((D, D), lambda b, m: (0, 0)),
            pl.BlockSpec((D, D), lambda b, m: (0, 0)),
            pl.BlockSpec((C, D), lambda b, m: (0, 0)),
            pl.BlockSpec((C, D), lambda b, m: (0, 0)),
            pl.BlockSpec((1, D, T * H), lambda b, m: (b, 0, 0)),
            pl.BlockSpec((1, MT, T), lambda b, m: (b, m, 0)),
            pl.BlockSpec((D, D), lambda b, m: (0, 0)),
            pl.BlockSpec((1, D), lambda b, m: (0, 0)),
        ],
        out_specs=pl.BlockSpec((1, T, D), lambda b, m: (b, 0, 0)),
        out_shape=jax.ShapeDtypeStruct((B, T, D), f32),
        scratch_shapes=[pltpu.VMEM((T, D), f32)],
        compiler_params=pltpu.CompilerParams(
            dimension_semantics=("arbitrary", "arbitrary")),
        interpret=_INTERPRET,
    )(contents, WkiT, WviT, posk, posv,
      q8T, swT, woT, out_b.reshape(1, D))

    return out


# speedup vs baseline: 1.3268x; 1.0082x over previous
import functools
import math

import jax
import jax.numpy as jnp
from jax import lax
from jax.experimental import pallas as pl
from jax.experimental.pallas import tpu as pltpu
from jax.experimental.pallas import tpu_sc as plsc

_INTERPRET = False

K_TOP = 8
H = 8


def _qk_body(q_ref, kT_ref, wqT_ref, wk_ref, wqiT_ref, bqi_ref, st_ref,
             pos_ref, wkiT_ref, wviT_ref, bki_ref, bvi_ref,
             logits_ref, q8T_ref, posk_ref, posv_ref,
             *, inv_sqrt_d, inv_sqrt_dh):
    q = q_ref[0]
    qh = jnp.dot(q, wqT_ref[...], preferred_element_type=jnp.float32)
    khT = jnp.dot(wk_ref[...], kT_ref[0], preferred_element_type=jnp.float32)
    logits_ref[0] = jnp.dot(qh, khT, preferred_element_type=jnp.float32) * inv_sqrt_d
    q2 = jnp.dot(qh, wqiT_ref[...], preferred_element_type=jnp.float32) + bqi_ref[...]
    T = q2.shape[0]
    q8 = (q2 * inv_sqrt_dh)[:, None, :] * st_ref[...][None, :, :]
    q8T_ref[0] = jnp.transpose(q8.reshape(T * H, q2.shape[1]))
    posk_ref[...] = jnp.dot(pos_ref[...], wkiT_ref[...],
                            preferred_element_type=jnp.float32) + bki_ref[...]
    posv_ref[...] = jnp.dot(pos_ref[...], wviT_ref[...],
                            preferred_element_type=jnp.float32) + bvi_ref[...]


def _fused_body(c_ref, wkiT_ref, wviT_ref, posk_ref, posv_ref,
                q8T_ref, swT_ref, woT_ref, bo_ref,
                out_ref, ctx_scr, *, mt, t_len, n_mt):
    m = pl.program_id(1)
    CH, C, D = c_ref.shape[1], c_ref.shape[2], c_ref.shape[3]
    T = t_len
    dh = D // H
    x = c_ref[0].reshape(CH * C, D)
    k2 = (jnp.dot(x, wkiT_ref[...], preferred_element_type=jnp.float32)
          .reshape(CH, C, D) + posk_ref[...][None, :, :])
    v2 = (jnp.dot(x, wviT_ref[...], preferred_element_type=jnp.float32)
          .reshape(CH, C, D) + posv_ref[...][None, :, :])
    sc = jnp.dot(k2.reshape(CH * C, D), q8T_ref[0],
                 preferred_element_type=jnp.float32)
    scr = sc.reshape(CH, C, T * H)
    mx = jnp.max(scr, axis=1, keepdims=True)
    e = jnp.exp(scr - mx)
    attn = e / jnp.sum(e, axis=1, keepdims=True)
    allo = lax.dot_general(attn, v2, (((1,), (1,)), ((0,), (0,))),
                           preferred_element_type=jnp.float32)
    alr = allo.reshape(CH, T, H, D)
    o_tile = jnp.concatenate(
        [alr[:, :, h, h * dh:(h + 1) * dh] for h in range(H)], axis=2)
    swm = swT_ref[0]
    contrib = jnp.sum(o_tile * swm[:, :, None], axis=0)

    @pl.when(m == 0)
    def _():
        ctx_scr[...] = jnp.zeros_like(ctx_scr)

    ctx_scr[...] += contrib

    @pl.when(m == n_mt - 1)
    def _():
        out_ref[0] = jnp.dot(ctx_scr[...], woT_ref[...],
                             preferred_element_type=jnp.float32) + bo_ref[...]


def _lane_shuffle(x, perm):
    return lax.gather(
        x, perm[:, None],
        dimension_numbers=lax.GatherDimensionNumbers(
            offset_dims=(), collapsed_slice_dims=(0,), start_index_map=(0,)),
        slice_sizes=(1,),
        mode=lax.GatherScatterMode.PROMISE_IN_BOUNDS)


def _lane_reduce(x, op, lane):
    for shift in (1, 2, 4, 8):
        perm = jnp.bitwise_and(lane + shift, 15)
        x = op(x, _lane_shuffle(x, perm))
    return x


def _sc_topk(logits2d, n_rows, n_mem, k_top):
    NC, NS = 2, 16
    NW = NC * NS
    rpw = n_rows // NW
    nv = n_mem // 16
    mesh = plsc.VectorSubcoreMesh(core_axis_name="c", subcore_axis_name="s")

    @functools.partial(
        pl.kernel,
        mesh=mesh,
        out_type=jax.ShapeDtypeStruct((n_rows, n_mem), jnp.float32),
        scratch_types=[pltpu.VMEM((n_mem,), jnp.float32),
                       pltpu.VMEM((n_mem,), jnp.float32)],
    )
    def sc_body(logits_hbm, selw_hbm, lrow_v, sel_v):
        wid = lax.axis_index("s") * NC + lax.axis_index("c")
        lane = lax.broadcasted_iota(jnp.int32, (16,), 0)
        zero16 = jnp.zeros((16,), jnp.float32)
        NEG = jnp.float32(-3.0e38)
        for r in range(rpw):
            row = wid * rpw + r
            pltpu.sync_copy(logits_hbm.at[row], lrow_v)
            vals = [lrow_v[pl.ds(j * 16, 16)] for j in range(nv)]
            idxs = [lane + j * 16 for j in range(nv)]
            topv = jnp.full((16,), NEG, jnp.float32)
            topi = jnp.zeros((16,), jnp.int32)
            for kk in range(k_top):
                m = vals[0]
                for j in range(1, nv):
                    m = jnp.maximum(m, vals[j])
                mx = _lane_reduce(m, jnp.maximum, lane)
                cand = jnp.full((16,), jnp.int32(n_mem * 2), jnp.int32)
                for j in range(nv):
                    cand = jnp.minimum(
                        cand, jnp.where(vals[j] == mx, idxs[j],
                                        jnp.int32(n_mem * 2)))
                ami = _lane_reduce(cand, jnp.minimum, lane)
                topv = jnp.where(lane == kk, mx, topv)
                topi = jnp.where(lane == kk, ami, topi)
                for j in range(nv):
                    vals[j] = jnp.where(idxs[j] == ami, NEG, vals[j])
            mall = _lane_reduce(topv, jnp.maximum, lane)
            e = jnp.exp(topv - mall)
            e = jnp.where(lane < k_top, e, jnp.float32(0.0))
            wrow = e / _lane_reduce(e, jnp.add, lane)
            sel = [zero16] * nv
            for kk in range(k_top):
                pk = jnp.full((16,), kk, jnp.int32)
                tb = _lane_shuffle(topi, pk)
                wb = _lane_shuffle(wrow, pk)
                for j in range(nv):
                    sel[j] = jnp.where(idxs[j] == tb, wb, sel[j])
            for j in range(nv):
                sel_v[pl.ds(j * 16, 16)] = sel[j]
            pltpu.sync_copy(sel_v, selw_hbm.at[row])

    return sc_body(logits2d)


def kernel(queries, keys, contents, steps_since_last_write, accumulator,
           Wq, Wk, Wv, in_proj_w, in_proj_b, out_w, out_b):
    B, T, D = queries.shape
    _, M, C, _ = contents.shape
    dh = D // H
    BT = B * T
    f32 = jnp.float32

    Wqi, Wki, Wvi = jnp.split(in_proj_w, 3, axis=0)
    bqi, bki, bvi = jnp.split(in_proj_b, 3)
    WqT, WqiT = Wq.T, Wqi.T
    WkiT, WviT = Wki.T, Wvi.T
    woT = out_w.T
    keysT = keys.transpose(0, 2, 1)

    freqs = jnp.arange(0.0, D, 2.0)
    inv_freq = 10000.0 ** (-freqs / D)
    pos_seq = jnp.arange(C - 1.0, -1.0, -1.0)
    sinusoid = pos_seq[:, None] * inv_freq[None, :]
    pos = jnp.concatenate([jnp.sin(sinusoid), jnp.cos(sinusoid)], axis=-1)

    MT = 32
    n_mt = M // MT

    S = (jnp.arange(D)[:, None] // dh == jnp.arange(H)[None, :]).astype(f32)
    ST = S.T

    logits, q8T, posk, posv = pl.pallas_call(
        functools.partial(_qk_body, inv_sqrt_d=1.0 / math.sqrt(D),
                          inv_sqrt_dh=1.0 / math.sqrt(dh)),
        grid=(B,),
        in_specs=[
            pl.BlockSpec((1, T, D), lambda b: (b, 0, 0)),
            pl.BlockSpec((1, D, M), lambda b: (b, 0, 0)),
            pl.BlockSpec((D, D), lambda b: (0, 0)),
            pl.BlockSpec((D, D), lambda b: (0, 0)),
            pl.BlockSpec((D, D), lambda b: (0, 0)),
            pl.BlockSpec((1, D), lambda b: (0, 0)),
            pl.BlockSpec((H, D), lambda b: (0, 0)),
            pl.BlockSpec((C, D), lambda b: (0, 0)),
            pl.BlockSpec((D, D), lambda b: (0, 0)),
            pl.BlockSpec((D, D), lambda b: (0, 0)),
            pl.BlockSpec((1, D), lambda b: (0, 0)),
            pl.BlockSpec((1, D), lambda b: (0, 0)),
        ],
        out_specs=[
            pl.BlockSpec((1, T, M), lambda b: (b, 0, 0)),
            pl.BlockSpec((1, D, T * H), lambda b: (b, 0, 0)),
            pl.BlockSpec((C, D), lambda b: (0, 0)),
            pl.BlockSpec((C, D), lambda b: (0, 0)),
        ],
        out_shape=[
            jax.ShapeDtypeStruct((B, T, M), f32),
            jax.ShapeDtypeStruct((B, D, T * H), f32),
            jax.ShapeDtypeStruct((C, D), f32),
            jax.ShapeDtypeStruct((C, D), f32),
        ],
        compiler_params=pltpu.CompilerParams(
            dimension_semantics=("arbitrary",)),
        interpret=_INTERPRET,
    )(queries, keysT, WqT, Wk, WqiT, bqi.reshape(1, D), ST, pos,
      WkiT, WviT, bki.reshape(1, D), bvi.reshape(1, D))

    selw = _sc_topk(logits.reshape(BT, M), BT, M, K_TOP)
    swT = selw.reshape(B, T, M).transpose(0, 2, 1)

    out = pl.pallas_call(
        functools.partial(_fused_body, mt=MT, t_len=T, n_mt=n_mt),
        grid=(B, n_mt),
        in_specs=[
            pl.BlockSpec((1, MT, C, D), lambda b, m: (b, m, 0, 0)),
            pl.BlockSpec((D, D), lambda b, m: (0, 0)),
            pl.BlockSpec((D, D), lambda b, m: (0, 0)),
            pl.BlockSpec((C, D), lambda b, m: (0, 0)),
            pl.BlockSpec((C, D), lambda b, m: (0, 0)),
            pl.BlockSpec((1, D, T * H), lambda b, m: (b, 0, 0)),
            pl.BlockSpec((1, MT, T), lambda b, m: (b, m, 0)),
            pl.BlockSpec((D, D), lambda b, m: (0, 0)),
            pl.BlockSpec((1, D), lambda b, m: (0, 0)),
        ],
        out_specs=pl.BlockSpec((1, T, D), lambda b, m: (b, 0, 0)),
        out_shape=jax.ShapeDtypeStruct((B, T, D), f32),
        scratch_shapes=[pltpu.VMEM((T, D), f32)],
        compiler_params=pltpu.CompilerParams(
            dimension_semantics=("arbitrary", "arbitrary")),
        interpret=_INTERPRET,
    )(contents, WkiT, WviT, posk, posv,
      q8T, swT, woT, out_b.reshape(1, D))

    return out


# speedup vs baseline: 1.4575x; 1.0984x over previous
import functools
import math

import jax
import jax.numpy as jnp
from jax import lax
from jax.experimental import pallas as pl
from jax.experimental.pallas import tpu as pltpu
from jax.experimental.pallas import tpu_sc as plsc

_INTERPRET = False

K_TOP = 8
H = 8


def _qk_body(q_ref, kT_ref, wqT_ref, wk_ref, wqiT_ref, bqi_ref, st_ref,
             pos_ref, wkiT_ref, wviT_ref, bki_ref, bvi_ref,
             logits_ref, q8T_ref, posk_ref, posv_ref,
             *, inv_sqrt_d, inv_sqrt_dh):
    q = q_ref[0]
    qh = jnp.dot(q, wqT_ref[...], preferred_element_type=jnp.float32)
    khT = jnp.dot(wk_ref[...], kT_ref[0], preferred_element_type=jnp.float32)
    logits_ref[0] = jnp.dot(qh, khT, preferred_element_type=jnp.float32) * inv_sqrt_d
    q2 = jnp.dot(qh, wqiT_ref[...], preferred_element_type=jnp.float32) + bqi_ref[...]
    T = q2.shape[0]
    q8 = st_ref[...][:, None, :] * (q2 * inv_sqrt_dh)[None, :, :]
    q8T_ref[0] = jnp.transpose(q8.reshape(H * T, q2.shape[1]))
    posk_ref[...] = jnp.dot(pos_ref[...], wkiT_ref[...],
                            preferred_element_type=jnp.float32) + bki_ref[...]
    posv_ref[...] = jnp.dot(pos_ref[...], wviT_ref[...],
                            preferred_element_type=jnp.float32) + bvi_ref[...]


def _fused_body(c_ref, wkiT_ref, wviT_ref, posk_ref, posv_ref,
                q8T_ref, swT_ref, woT_ref, bo_ref,
                out_ref, ctx_scr, *, mt, t_len, n_mt):
    m = pl.program_id(1)
    CH, C, D = c_ref.shape[1], c_ref.shape[2], c_ref.shape[3]
    T = t_len
    dh = D // H
    x = c_ref[0].reshape(CH * C, D)
    k2 = (jnp.dot(x, wkiT_ref[...], preferred_element_type=jnp.float32)
          .reshape(CH, C, D) + posk_ref[...][None, :, :])
    v2 = (jnp.dot(x, wviT_ref[...], preferred_element_type=jnp.float32)
          .reshape(CH, C, D) + posv_ref[...][None, :, :])
    sc = jnp.dot(k2.reshape(CH * C, D), q8T_ref[0],
                 preferred_element_type=jnp.float32)
    scr = sc.reshape(CH, C, H * T)
    mx = jnp.max(scr, axis=1, keepdims=True)
    e = jnp.exp(scr - mx)
    attn = e / jnp.sum(e, axis=1, keepdims=True)
    swexp = jnp.tile(swT_ref[0], (1, H))
    attn_w = attn * swexp[:, None, :]
    part = lax.dot_general(attn_w.reshape(CH * C, H * T),
                           v2.reshape(CH * C, D),
                           (((0,), (0,)), ((), ())),
                           preferred_element_type=jnp.float32)

    @pl.when(m == 0)
    def _():
        ctx_scr[...] = jnp.zeros_like(ctx_scr)

    ctx_scr[...] += part

    @pl.when(m == n_mt - 1)
    def _():
        ctx = jnp.concatenate(
            [ctx_scr[h * T:(h + 1) * T, h * dh:(h + 1) * dh]
             for h in range(H)], axis=1)
        out_ref[0] = jnp.dot(ctx, woT_ref[...],
                             preferred_element_type=jnp.float32) + bo_ref[...]


def _lane_shuffle(x, perm):
    return lax.gather(
        x, perm[:, None],
        dimension_numbers=lax.GatherDimensionNumbers(
            offset_dims=(), collapsed_slice_dims=(0,), start_index_map=(0,)),
        slice_sizes=(1,),
        mode=lax.GatherScatterMode.PROMISE_IN_BOUNDS)


def _lane_reduce(x, op, lane):
    for shift in (1, 2, 4, 8):
        perm = jnp.bitwise_and(lane + shift, 15)
        x = op(x, _lane_shuffle(x, perm))
    return x


def _sc_topk(logits2d, n_rows, n_mem, k_top):
    NC, NS = 2, 16
    NW = NC * NS
    rpw = n_rows // NW
    nv = n_mem // 16
    mesh = plsc.VectorSubcoreMesh(core_axis_name="c", subcore_axis_name="s")

    @functools.partial(
        pl.kernel,
        mesh=mesh,
        out_type=jax.ShapeDtypeStruct((n_rows, n_mem), jnp.float32),
        scratch_types=[pltpu.VMEM((n_mem,), jnp.float32),
                       pltpu.VMEM((n_mem,), jnp.float32)],
    )
    def sc_body(logits_hbm, selw_hbm, lrow_v, sel_v):
        wid = lax.axis_index("s") * NC + lax.axis_index("c")
        lane = lax.broadcasted_iota(jnp.int32, (16,), 0)
        zero16 = jnp.zeros((16,), jnp.float32)
        NEG = jnp.float32(-3.0e38)
        for r in range(rpw):
            row = wid * rpw + r
            pltpu.sync_copy(logits_hbm.at[row], lrow_v)
            vals = [lrow_v[pl.ds(j * 16, 16)] for j in range(nv)]
            idxs = [lane + j * 16 for j in range(nv)]
            topv = jnp.full((16,), NEG, jnp.float32)
            topi = jnp.zeros((16,), jnp.int32)
            for kk in range(k_top):
                m = vals[0]
                for j in range(1, nv):
                    m = jnp.maximum(m, vals[j])
                mx = _lane_reduce(m, jnp.maximum, lane)
                cand = jnp.full((16,), jnp.int32(n_mem * 2), jnp.int32)
                for j in range(nv):
                    cand = jnp.minimum(
                        cand, jnp.where(vals[j] == mx, idxs[j],
                                        jnp.int32(n_mem * 2)))
                ami = _lane_reduce(cand, jnp.minimum, lane)
                topv = jnp.where(lane == kk, mx, topv)
                topi = jnp.where(lane == kk, ami, topi)
                for j in range(nv):
                    vals[j] = jnp.where(idxs[j] == ami, NEG, vals[j])
            mall = _lane_reduce(topv, jnp.maximum, lane)
            e = jnp.exp(topv - mall)
            e = jnp.where(lane < k_top, e, jnp.float32(0.0))
            wrow = e / _lane_reduce(e, jnp.add, lane)
            sel = [zero16] * nv
            for kk in range(k_top):
                pk = jnp.full((16,), kk, jnp.int32)
                tb = _lane_shuffle(topi, pk)
                wb = _lane_shuffle(wrow, pk)
                for j in range(nv):
                    sel[j] = jnp.where(idxs[j] == tb, wb, sel[j])
            for j in range(nv):
                sel_v[pl.ds(j * 16, 16)] = sel[j]
            pltpu.sync_copy(sel_v, selw_hbm.at[row])

    return sc_body(logits2d)


def kernel(queries, keys, contents, steps_since_last_write, accumulator,
           Wq, Wk, Wv, in_proj_w, in_proj_b, out_w, out_b):
    B, T, D = queries.shape
    _, M, C, _ = contents.shape
    dh = D // H
    BT = B * T
    f32 = jnp.float32

    Wqi, Wki, Wvi = jnp.split(in_proj_w, 3, axis=0)
    bqi, bki, bvi = jnp.split(in_proj_b, 3)
    WqT, WqiT = Wq.T, Wqi.T
    WkiT, WviT = Wki.T, Wvi.T
    woT = out_w.T
    keysT = keys.transpose(0, 2, 1)

    freqs = jnp.arange(0.0, D, 2.0)
    inv_freq = 10000.0 ** (-freqs / D)
    pos_seq = jnp.arange(C - 1.0, -1.0, -1.0)
    sinusoid = pos_seq[:, None] * inv_freq[None, :]
    pos = jnp.concatenate([jnp.sin(sinusoid), jnp.cos(sinusoid)], axis=-1)

    MT = 32
    n_mt = M // MT

    S = (jnp.arange(D)[:, None] // dh == jnp.arange(H)[None, :]).astype(f32)
    ST = S.T

    logits, q8T, posk, posv = pl.pallas_call(
        functools.partial(_qk_body, inv_sqrt_d=1.0 / math.sqrt(D),
                          inv_sqrt_dh=1.0 / math.sqrt(dh)),
        grid=(B,),
        in_specs=[
            pl.BlockSpec((1, T, D), lambda b: (b, 0, 0)),
            pl.BlockSpec((1, D, M), lambda b: (b, 0, 0)),
            pl.BlockSpec((D, D), lambda b: (0, 0)),
            pl.BlockSpec((D, D), lambda b: (0, 0)),
            pl.BlockSpec((D, D), lambda b: (0, 0)),
            pl.BlockSpec((1, D), lambda b: (0, 0)),
            pl.BlockSpec((H, D), lambda b: (0, 0)),
            pl.BlockSpec((C, D), lambda b: (0, 0)),
            pl.BlockSpec((D, D), lambda b: (0, 0)),
            pl.BlockSpec((D, D), lambda b: (0, 0)),
            pl.BlockSpec((1, D), lambda b: (0, 0)),
            pl.BlockSpec((1, D), lambda b: (0, 0)),
        ],
        out_specs=[
            pl.BlockSpec((1, T, M), lambda b: (b, 0, 0)),
            pl.BlockSpec((1, D, T * H), lambda b: (b, 0, 0)),
            pl.BlockSpec((C, D), lambda b: (0, 0)),
            pl.BlockSpec((C, D), lambda b: (0, 0)),
        ],
        out_shape=[
            jax.ShapeDtypeStruct((B, T, M), f32),
            jax.ShapeDtypeStruct((B, D, T * H), f32),
            jax.ShapeDtypeStruct((C, D), f32),
            jax.ShapeDtypeStruct((C, D), f32),
        ],
        compiler_params=pltpu.CompilerParams(
            dimension_semantics=("arbitrary",)),
        interpret=_INTERPRET,
    )(queries, keysT, WqT, Wk, WqiT, bqi.reshape(1, D), ST, pos,
      WkiT, WviT, bki.reshape(1, D), bvi.reshape(1, D))

    selw = _sc_topk(logits.reshape(BT, M), BT, M, K_TOP)
    swT = selw.reshape(B, T, M).transpose(0, 2, 1)

    out = pl.pallas_call(
        functools.partial(_fused_body, mt=MT, t_len=T, n_mt=n_mt),
        grid=(B, n_mt),
        in_specs=[
            pl.BlockSpec((1, MT, C, D), lambda b, m: (b, m, 0, 0)),
            pl.BlockSpec((D, D), lambda b, m: (0, 0)),
            pl.BlockSpec((D, D), lambda b, m: (0, 0)),
            pl.BlockSpec((C, D), lambda b, m: (0, 0)),
            pl.BlockSpec((C, D), lambda b, m: (0, 0)),
            pl.BlockSpec((1, D, T * H), lambda b, m: (b, 0, 0)),
            pl.BlockSpec((1, MT, T), lambda b, m: (b, m, 0)),
            pl.BlockSpec((D, D), lambda b, m: (0, 0)),
            pl.BlockSpec((1, D), lambda b, m: (0, 0)),
        ],
        out_specs=pl.BlockSpec((1, T, D), lambda b, m: (b, 0, 0)),
        out_shape=jax.ShapeDtypeStruct((B, T, D), f32),
        scratch_shapes=[pltpu.VMEM((H * T, D), f32)],
        compiler_params=pltpu.CompilerParams(
            dimension_semantics=("arbitrary", "arbitrary")),
        interpret=_INTERPRET,
    )(contents, WkiT, WviT, posk, posv,
      q8T, swT, woT, out_b.reshape(1, D))

    return out


# speedup vs baseline: 1.4747x; 1.0118x over previous
import functools
import math

import jax
import jax.numpy as jnp
from jax import lax
from jax.experimental import pallas as pl
from jax.experimental.pallas import tpu as pltpu
from jax.experimental.pallas import tpu_sc as plsc

_INTERPRET = False

K_TOP = 8
H = 8


def _qk_body(q_ref, kT_ref, wqT_ref, wk_ref, wqiT_ref, bqi_ref, st_ref,
             pos_ref, wkiT_ref, wviT_ref, bki_ref, bvi_ref,
             logits_ref, q8T_ref, posk_ref, posv_ref,
             *, inv_sqrt_d, inv_sqrt_dh):
    q = q_ref[0]
    qh = jnp.dot(q, wqT_ref[...], preferred_element_type=jnp.float32)
    khT = jnp.dot(wk_ref[...], kT_ref[0], preferred_element_type=jnp.float32)
    logits_ref[0] = jnp.dot(qh, khT, preferred_element_type=jnp.float32) * inv_sqrt_d
    q2 = jnp.dot(qh, wqiT_ref[...], preferred_element_type=jnp.float32) + bqi_ref[...]
    T = q2.shape[0]
    q8 = st_ref[...][:, None, :] * (q2 * inv_sqrt_dh)[None, :, :]
    q8T_ref[0] = jnp.transpose(q8.reshape(H * T, q2.shape[1]))
    posk_ref[...] = jnp.dot(pos_ref[...], wkiT_ref[...],
                            preferred_element_type=jnp.float32) + bki_ref[...]
    posv_ref[...] = jnp.dot(pos_ref[...], wviT_ref[...],
                            preferred_element_type=jnp.float32) + bvi_ref[...]


def _fused_body(c_ref, wkiT_ref, wviT_ref, posk_ref, posv_ref,
                q8T_ref, swT_ref, woT_ref, bo_ref,
                out_ref, ctx_scr, *, mt, t_len, n_mt):
    m = pl.program_id(1)
    CH, C, D = c_ref.shape[1], c_ref.shape[2], c_ref.shape[3]
    T = t_len
    dh = D // H
    x = c_ref[0].reshape(CH * C, D)
    k2 = (jnp.dot(x, wkiT_ref[...], preferred_element_type=jnp.float32)
          .reshape(CH, C, D) + posk_ref[...][None, :, :])
    v2 = (jnp.dot(x, wviT_ref[...], preferred_element_type=jnp.float32)
          .reshape(CH, C, D) + posv_ref[...][None, :, :])
    sc = jnp.dot(k2.reshape(CH * C, D), q8T_ref[0],
                 preferred_element_type=jnp.float32)
    scr = sc.reshape(CH, C, H * T)
    mx = jnp.max(scr, axis=1, keepdims=True)
    e = jnp.exp(scr - mx)
    attn = e / jnp.sum(e, axis=1, keepdims=True)
    swexp = jnp.tile(swT_ref[0], (1, H))
    attn_w = attn * swexp[:, None, :]
    part = lax.dot_general(attn_w.reshape(CH * C, H * T),
                           v2.reshape(CH * C, D),
                           (((0,), (0,)), ((), ())),
                           preferred_element_type=jnp.float32)

    @pl.when(m == 0)
    def _():
        ctx_scr[...] = jnp.zeros_like(ctx_scr)

    ctx_scr[...] += part

    @pl.when(m == n_mt - 1)
    def _():
        ctx = jnp.concatenate(
            [ctx_scr[h * T:(h + 1) * T, h * dh:(h + 1) * dh]
             for h in range(H)], axis=1)
        out_ref[0] = jnp.dot(ctx, woT_ref[...],
                             preferred_element_type=jnp.float32) + bo_ref[...]


def _lane_shuffle(x, perm):
    return lax.gather(
        x, perm[:, None],
        dimension_numbers=lax.GatherDimensionNumbers(
            offset_dims=(), collapsed_slice_dims=(0,), start_index_map=(0,)),
        slice_sizes=(1,),
        mode=lax.GatherScatterMode.PROMISE_IN_BOUNDS)


def _lane_reduce(x, op, lane):
    for shift in (1, 2, 4, 8):
        perm = jnp.bitwise_and(lane + shift, 15)
        x = op(x, _lane_shuffle(x, perm))
    return x


def _sc_topk(logits2d, n_rows, n_mem, k_top):
    NC, NS = 2, 16
    NW = NC * NS
    rpw = n_rows // NW
    nv = n_mem // 16
    mesh = plsc.VectorSubcoreMesh(core_axis_name="c", subcore_axis_name="s")

    @functools.partial(
        pl.kernel,
        mesh=mesh,
        out_type=jax.ShapeDtypeStruct((n_rows, n_mem), jnp.float32),
        scratch_types=[pltpu.VMEM((n_mem,), jnp.float32),
                       pltpu.VMEM((n_mem,), jnp.float32)],
    )
    def sc_body(logits_hbm, selw_hbm, lrow_v, sel_v):
        wid = lax.axis_index("s") * NC + lax.axis_index("c")
        lane = lax.broadcasted_iota(jnp.int32, (16,), 0)
        zero16 = jnp.zeros((16,), jnp.float32)
        NEG = jnp.float32(-3.0e38)
        for r in range(rpw):
            row = wid * rpw + r
            pltpu.sync_copy(logits_hbm.at[row], lrow_v)
            vals = [lrow_v[pl.ds(j * 16, 16)] for j in range(nv)]
            idxs = [lane + j * 16 for j in range(nv)]
            topv = jnp.full((16,), NEG, jnp.float32)
            topi = jnp.zeros((16,), jnp.int32)
            for kk in range(k_top):
                m = vals[0]
                for j in range(1, nv):
                    m = jnp.maximum(m, vals[j])
                mx = _lane_reduce(m, jnp.maximum, lane)
                cand = jnp.full((16,), jnp.int32(n_mem * 2), jnp.int32)
                for j in range(nv):
                    cand = jnp.minimum(
                        cand, jnp.where(vals[j] == mx, idxs[j],
                                        jnp.int32(n_mem * 2)))
                ami = _lane_reduce(cand, jnp.minimum, lane)
                topv = jnp.where(lane == kk, mx, topv)
                topi = jnp.where(lane == kk, ami, topi)
                for j in range(nv):
                    vals[j] = jnp.where(idxs[j] == ami, NEG, vals[j])
            mall = _lane_reduce(topv, jnp.maximum, lane)
            e = jnp.exp(topv - mall)
            e = jnp.where(lane < k_top, e, jnp.float32(0.0))
            wrow = e / _lane_reduce(e, jnp.add, lane)
            sel = [zero16] * nv
            for kk in range(k_top):
                pk = jnp.full((16,), kk, jnp.int32)
                tb = _lane_shuffle(topi, pk)
                wb = _lane_shuffle(wrow, pk)
                for j in range(nv):
                    sel[j] = jnp.where(idxs[j] == tb, wb, sel[j])
            for j in range(nv):
                sel_v[pl.ds(j * 16, 16)] = sel[j]
            pltpu.sync_copy(sel_v, selw_hbm.at[row])

    return sc_body(logits2d)


def kernel(queries, keys, contents, steps_since_last_write, accumulator,
           Wq, Wk, Wv, in_proj_w, in_proj_b, out_w, out_b):
    B, T, D = queries.shape
    _, M, C, _ = contents.shape
    dh = D // H
    BT = B * T
    f32 = jnp.float32

    Wqi, Wki, Wvi = jnp.split(in_proj_w, 3, axis=0)
    bqi, bki, bvi = jnp.split(in_proj_b, 3)
    WqT, WqiT = Wq.T, Wqi.T
    WkiT, WviT = Wki.T, Wvi.T
    woT = out_w.T
    keysT = keys.transpose(0, 2, 1)

    freqs = jnp.arange(0.0, D, 2.0)
    inv_freq = 10000.0 ** (-freqs / D)
    pos_seq = jnp.arange(C - 1.0, -1.0, -1.0)
    sinusoid = pos_seq[:, None] * inv_freq[None, :]
    pos = jnp.concatenate([jnp.sin(sinusoid), jnp.cos(sinusoid)], axis=-1)

    MT = 64
    n_mt = M // MT

    S = (jnp.arange(D)[:, None] // dh == jnp.arange(H)[None, :]).astype(f32)
    ST = S.T

    logits, q8T, posk, posv = pl.pallas_call(
        functools.partial(_qk_body, inv_sqrt_d=1.0 / math.sqrt(D),
                          inv_sqrt_dh=1.0 / math.sqrt(dh)),
        grid=(B,),
        in_specs=[
            pl.BlockSpec((1, T, D), lambda b: (b, 0, 0)),
            pl.BlockSpec((1, D, M), lambda b: (b, 0, 0)),
            pl.BlockSpec((D, D), lambda b: (0, 0)),
            pl.BlockSpec((D, D), lambda b: (0, 0)),
            pl.BlockSpec((D, D), lambda b: (0, 0)),
            pl.BlockSpec((1, D), lambda b: (0, 0)),
            pl.BlockSpec((H, D), lambda b: (0, 0)),
            pl.BlockSpec((C, D), lambda b: (0, 0)),
            pl.BlockSpec((D, D), lambda b: (0, 0)),
            pl.BlockSpec((D, D), lambda b: (0, 0)),
            pl.BlockSpec((1, D), lambda b: (0, 0)),
            pl.BlockSpec((1, D), lambda b: (0, 0)),
        ],
        out_specs=[
            pl.BlockSpec((1, T, M), lambda b: (b, 0, 0)),
            pl.BlockSpec((1, D, T * H), lambda b: (b, 0, 0)),
            pl.BlockSpec((C, D), lambda b: (0, 0)),
            pl.BlockSpec((C, D), lambda b: (0, 0)),
        ],
        out_shape=[
            jax.ShapeDtypeStruct((B, T, M), f32),
            jax.ShapeDtypeStruct((B, D, T * H), f32),
            jax.ShapeDtypeStruct((C, D), f32),
            jax.ShapeDtypeStruct((C, D), f32),
        ],
        compiler_params=pltpu.CompilerParams(
            dimension_semantics=("arbitrary",)),
        interpret=_INTERPRET,
    )(queries, keysT, WqT, Wk, WqiT, bqi.reshape(1, D), ST, pos,
      WkiT, WviT, bki.reshape(1, D), bvi.reshape(1, D))

    selw = _sc_topk(logits.reshape(BT, M), BT, M, K_TOP)
    swT = selw.reshape(B, T, M).transpose(0, 2, 1)

    out = pl.pallas_call(
        functools.partial(_fused_body, mt=MT, t_len=T, n_mt=n_mt),
        grid=(B, n_mt),
        in_specs=[
            pl.BlockSpec((1, MT, C, D), lambda b, m: (b, m, 0, 0)),
            pl.BlockSpec((D, D), lambda b, m: (0, 0)),
            pl.BlockSpec((D, D), lambda b, m: (0, 0)),
            pl.BlockSpec((C, D), lambda b, m: (0, 0)),
            pl.BlockSpec((C, D), lambda b, m: (0, 0)),
            pl.BlockSpec((1, D, T * H), lambda b, m: (b, 0, 0)),
            pl.BlockSpec((1, MT, T), lambda b, m: (b, m, 0)),
            pl.BlockSpec((D, D), lambda b, m: (0, 0)),
            pl.BlockSpec((1, D), lambda b, m: (0, 0)),
        ],
        out_specs=pl.BlockSpec((1, T, D), lambda b, m: (b, 0, 0)),
        out_shape=jax.ShapeDtypeStruct((B, T, D), f32),
        scratch_shapes=[pltpu.VMEM((H * T, D), f32)],
        compiler_params=pltpu.CompilerParams(
            dimension_semantics=("arbitrary", "arbitrary")),
        interpret=_INTERPRET,
    )(contents, WkiT, WviT, posk, posv,
      q8T, swT, woT, out_b.reshape(1, D))

    return out
